# baseline (speedup 1.0000x reference)
import functools

import jax
import jax.numpy as jnp
from jax.experimental import pallas as pl
from jax.experimental.pallas import tpu as pltpu

B, N, M, K = 16, 2048, 2048, 3
N_BLK = 512
NB = N // N_BLK


def _chamfer_kernel(x1_ref, x2t_ref, out_ref, cmin_ref, s1_ref, s2_ref):
    b = pl.program_id(0)
    n = pl.program_id(1)

    x1 = x1_ref[0]
    x2t = x2t_ref[0]

    dot = jnp.dot(x1, x2t, preferred_element_type=jnp.float32,
                  precision=jax.lax.Precision.HIGHEST)
    n1 = jnp.sum(x1 * x1, axis=1, keepdims=True)
    n2 = jnp.sum(x2t * x2t, axis=0, keepdims=True)
    dist = (n1 + n2) - 2.0 * dot

    row_min = jnp.min(dist, axis=1, keepdims=True)
    col_min = jnp.min(dist, axis=0, keepdims=True)

    @pl.when(jnp.logical_and(b == 0, n == 0))
    def _init():
        s1_ref[0] = 0.0
        s2_ref[0] = 0.0

    @pl.when(n == 0)
    def _init_cmin():
        cmin_ref[...] = col_min

    @pl.when(n != 0)
    def _acc_cmin():
        cmin_ref[...] = jnp.minimum(cmin_ref[...], col_min)

    s1_ref[0] += jnp.sum(row_min)

    @pl.when(n == NB - 1)
    def _finish_batch():
        s2_ref[0] += jnp.sum(cmin_ref[...])

    @pl.when(jnp.logical_and(b == B - 1, n == NB - 1))
    def _finish():
        out_ref[0, 0] = s1_ref[0] / (B * N) + s2_ref[0] / (B * M)


@jax.jit
def kernel(xyz1, xyz2):
    x2t = jnp.transpose(xyz2, (0, 2, 1))
    out = pl.pallas_call(
        _chamfer_kernel,
        grid=(B, NB),
        in_specs=[
            pl.BlockSpec((1, N_BLK, K), lambda b, n: (b, n, 0)),
            pl.BlockSpec((1, K, M), lambda b, n: (b, 0, 0)),
        ],
        out_specs=pl.BlockSpec(
            (1, 1), lambda b, n: (0, 0), memory_space=pltpu.SMEM
        ),
        out_shape=jax.ShapeDtypeStruct((1, 1), jnp.float32),
        scratch_shapes=[
            pltpu.VMEM((1, M), jnp.float32),
            pltpu.SMEM((1,), jnp.float32),
            pltpu.SMEM((1,), jnp.float32),
        ],
    )(xyz1, x2t)
    return out[0, 0]


# speedup vs baseline: 1.9364x; 1.9364x over previous
import jax
import jax.numpy as jnp
from jax.experimental import pallas as pl
from jax.experimental.pallas import tpu as pltpu

B, N, M, K = 16, 2048, 2048, 3
N_BLK = 512
NB = N // N_BLK


def _chamfer_kernel(x1_ref, x2t_ref, out_ref, cmin_ref, s1_ref, s2_ref):
    b = pl.program_id(0)
    n = pl.program_id(1)

    x1 = x1_ref[0]
    x2t = x2t_ref[0]

    a0 = x1[:, 0:1]
    a1 = x1[:, 1:2]
    a2 = x1[:, 2:3]
    b0 = x2t[0:1, :]
    b1 = x2t[1:2, :]
    b2 = x2t[2:3, :]

    d0 = a0 - b0
    dist = d0 * d0
    d1 = a1 - b1
    dist = d1 * d1 + dist
    d2 = a2 - b2
    dist = d2 * d2 + dist

    row_min = jnp.min(dist, axis=1, keepdims=True)
    col_min = jnp.min(dist, axis=0, keepdims=True)

    @pl.when(jnp.logical_and(b == 0, n == 0))
    def _init():
        s1_ref[0] = 0.0
        s2_ref[0] = 0.0

    @pl.when(n == 0)
    def _init_cmin():
        cmin_ref[...] = col_min

    @pl.when(n != 0)
    def _acc_cmin():
        cmin_ref[...] = jnp.minimum(cmin_ref[...], col_min)

    s1_ref[0] += jnp.sum(row_min)

    @pl.when(n == NB - 1)
    def _finish_batch():
        s2_ref[0] += jnp.sum(cmin_ref[...])

    @pl.when(jnp.logical_and(b == B - 1, n == NB - 1))
    def _finish():
        out_ref[0, 0] = s1_ref[0] / (B * N) + s2_ref[0] / (B * M)


@jax.jit
def kernel(xyz1, xyz2):
    x2t = jnp.transpose(xyz2, (0, 2, 1))
    out = pl.pallas_call(
        _chamfer_kernel,
        grid=(B, NB),
        in_specs=[
            pl.BlockSpec((1, N_BLK, K), lambda b, n: (b, n, 0)),
            pl.BlockSpec((1, K, M), lambda b, n: (b, 0, 0)),
        ],
        out_specs=pl.BlockSpec(
            (1, 1), lambda b, n: (0, 0), memory_space=pltpu.SMEM
        ),
        out_shape=jax.ShapeDtypeStruct((1, 1), jnp.float32),
        scratch_shapes=[
            pltpu.VMEM((1, M), jnp.float32),
            pltpu.SMEM((1,), jnp.float32),
            pltpu.SMEM((1,), jnp.float32),
        ],
    )(xyz1, x2t)
    return out[0, 0]


# speedup vs baseline: 2.2187x; 1.1458x over previous
import jax
import jax.numpy as jnp
from jax.experimental import pallas as pl
from jax.experimental.pallas import tpu as pltpu

B, N, M, K = 16, 2048, 2048, 3
N_BLK = 2048
NB = N // N_BLK


def _chamfer_kernel(x1_ref, x2t_ref, out_ref, cmin_ref, s1_ref, s2_ref):
    b = pl.program_id(0)
    n = pl.program_id(1)

    x1 = x1_ref[0]
    x2t = x2t_ref[0]

    a0 = x1[:, 0:1]
    a1 = x1[:, 1:2]
    a2 = x1[:, 2:3]
    b0 = x2t[0:1, :]
    b1 = x2t[1:2, :]
    b2 = x2t[2:3, :]

    d0 = a0 - b0
    dist = d0 * d0
    d1 = a1 - b1
    dist = d1 * d1 + dist
    d2 = a2 - b2
    dist = d2 * d2 + dist

    row_min = jnp.min(dist, axis=1, keepdims=True)
    col_min = jnp.min(dist, axis=0, keepdims=True)

    @pl.when(jnp.logical_and(b == 0, n == 0))
    def _init():
        s1_ref[0] = 0.0
        s2_ref[0] = 0.0

    @pl.when(n == 0)
    def _init_cmin():
        cmin_ref[...] = col_min

    @pl.when(n != 0)
    def _acc_cmin():
        cmin_ref[...] = jnp.minimum(cmin_ref[...], col_min)

    s1_ref[0] += jnp.sum(row_min)

    @pl.when(n == NB - 1)
    def _finish_batch():
        s2_ref[0] += jnp.sum(cmin_ref[...])

    @pl.when(jnp.logical_and(b == B - 1, n == NB - 1))
    def _finish():
        out_ref[0, 0] = s1_ref[0] / (B * N) + s2_ref[0] / (B * M)


@jax.jit
def kernel(xyz1, xyz2):
    x2t = jnp.transpose(xyz2, (0, 2, 1))
    out = pl.pallas_call(
        _chamfer_kernel,
        grid=(B, NB),
        in_specs=[
            pl.BlockSpec((1, N_BLK, K), lambda b, n: (b, n, 0)),
            pl.BlockSpec((1, K, M), lambda b, n: (b, 0, 0)),
        ],
        out_specs=pl.BlockSpec(
            (1, 1), lambda b, n: (0, 0), memory_space=pltpu.SMEM
        ),
        out_shape=jax.ShapeDtypeStruct((1, 1), jnp.float32),
        scratch_shapes=[
            pltpu.VMEM((1, M), jnp.float32),
            pltpu.SMEM((1,), jnp.float32),
            pltpu.SMEM((1,), jnp.float32),
        ],
    )(xyz1, x2t)
    return out[0, 0]
